# baseline (speedup 1.0000x reference)
import jax
import jax.numpy as jnp
from jax.experimental import pallas as pl
from jax.experimental.pallas import tpu as pltpu

_LIMIT = float(jnp.pi) / 2.0
_WEIGHT = 0.01

_N = 16384
_D = 72
_BANDS = (1, 3)
_BAND_ROWS = ((1, 2), (0,))


_COLS = (9, 10, 24)


def _tc_body(a_ref, b_ref, c_ref, o_ref):
    acc = jnp.float32(0.0)
    for ref in (a_ref, b_ref, c_ref):
        v = ref[...]
        p = jnp.where(jnp.abs(v) > _LIMIT, v, 0.0)
        acc = acc + jnp.sum(p * p)
    o_ref[0] = acc * _WEIGHT


@jax.jit
def kernel(pose):
    xt = pose.T.reshape(_D, _N // 128, 128)
    out = pl.pallas_call(
        _tc_body,
        grid=(1,),
        in_specs=[
            pl.BlockSpec((1, _N // 128, 128), lambda i, c=c: (c, 0, 0))
            for c in _COLS
        ],
        out_specs=pl.BlockSpec(memory_space=pltpu.SMEM),
        out_shape=jax.ShapeDtypeStruct((1,), jnp.float32),
    )(xt, xt, xt)
    return out[0]


# speedup vs baseline: 3.2063x; 3.2063x over previous
import jax
import jax.numpy as jnp
from jax.experimental import pallas as pl
from jax.experimental.pallas import tpu as pltpu

_LIMIT = float(jnp.pi) / 2.0
_LIMIT_SQ = _LIMIT * _LIMIT
_WEIGHT = 0.01

_N = 16384
_D = 72
_BANDS = (1, 3)
_BAND_ROWS = ((1, 2), (0,))


def _tc_body(a_ref, b_ref, o_ref):
    acc = jnp.float32(0.0)
    for ref, rows in zip((a_ref, b_ref), _BAND_ROWS):
        v = ref[...]
        r = jax.lax.broadcasted_iota(jnp.int32, v.shape, 0)
        keep = r == rows[0]
        for extra in rows[1:]:
            keep = keep | (r == extra)
        sq = v * v
        keep = keep & (sq > _LIMIT_SQ)
        acc = acc + jnp.sum(jnp.where(keep, sq, 0.0))
    o_ref[0] = acc * _WEIGHT


@jax.jit
def kernel(pose):
    xt = pose.T
    out = pl.pallas_call(
        _tc_body,
        grid=(1,),
        in_specs=[
            pl.BlockSpec((8, _N), lambda i, b=b: (b, 0)) for b in _BANDS
        ],
        out_specs=pl.BlockSpec(memory_space=pltpu.SMEM),
        out_shape=jax.ShapeDtypeStruct((1,), jnp.float32),
    )(xt, xt)
    return out[0]
